# baseline (speedup 1.0000x reference)
import jax
import jax.numpy as jnp
from jax.experimental import pallas as pl
from jax.experimental.pallas import tpu as pltpu

_BLOCK_ROWS = 1024
_STRIP = 1024


_SC_MIN = 3.0517578125e-05


def _pow2_ceil_scale(a):
    b = jax.lax.bitcast_convert_type(a, jnp.int32)
    p2 = jax.lax.bitcast_convert_type(
        (b + 0x7FFFFF) & jnp.int32(-8388608), jnp.float32)
    return jnp.clip(p2 * (1.0 / 128.0), _SC_MIN, 1.0)


def _pow2_recip(s):
    b = jax.lax.bitcast_convert_type(s, jnp.int32)
    return jax.lax.bitcast_convert_type(jnp.int32(0x7F000000) - b, jnp.float32)


def _strip_compute(x, u):
    ax = jnp.abs(x)
    scale = _pow2_ceil_scale(ax)
    m = jnp.clip(jnp.round(x * _pow2_recip(scale)), -128.0, 127.0)
    qx = m * scale

    new_mx = jnp.clip(jnp.floor(qx * 184.0), -1024.0, 1023.0) * (1 / 128.0)
    n = jnp.floor(new_mx)
    r = new_mx - n
    mexp = jnp.round(jnp.exp2(r) * 64.0)

    p = jnp.exp2(n * 11.0).astype(jnp.bfloat16)
    s = jnp.dot(p, u, preferred_element_type=jnp.float32)
    ebf = (jax.lax.bitcast_convert_type(s, jnp.int32) >> 23).astype(jnp.float32)
    M4 = jnp.floor((ebf - 170.5) * (1.0 / 11.0))
    m_fin = jnp.max(n, axis=1, keepdims=True)
    m_fin4 = m_fin - 4.0

    d1 = jnp.exp2(n - M4)
    d2 = jnp.exp2(M4 - m_fin4)
    t = jnp.floor(mexp * d1)
    v = t * d2
    th = jnp.floor(v)
    tl = v - th
    half = x.shape[1] // 2
    s_hi = jnp.sum(th, axis=1, keepdims=True)
    lo1 = jnp.sum(tl[:, :half], axis=1, keepdims=True)
    lo2 = jnp.sum(tl[:, half:], axis=1, keepdims=True)
    s_lo = ((lo1 * 32768.0).astype(jnp.int32)
            + (lo2 * 32768.0).astype(jnp.int32))
    mexp_sum = s_hi + (s_lo >> 15).astype(jnp.float32)

    rdiv = 4096.0 / mexp_sum
    mout = jnp.floor(mexp * rdiv)
    qout = mout * d1 * d2 * (1.0 / 4096.0)
    sc1 = _pow2_ceil_scale(qout)
    m1 = jnp.clip(jnp.round(qout * _pow2_recip(sc1)), -128.0, 127.0)
    q1 = m1 * sc1
    return jnp.where((m1 == 64.0) & (sc1 > _SC_MIN), q1 * 0.9921875, q1)


def _softmax_body(x_ref, u_ref, o_ref):
    u = u_ref[...]
    for i in range(_BLOCK_ROWS // _STRIP):
        sl = slice(i * _STRIP, (i + 1) * _STRIP)
        o_ref[sl, :] = _strip_compute(x_ref[sl, :], u)


def kernel(x):
    n_rows, n_feat = x.shape
    u = jnp.triu(jnp.ones((n_feat, n_feat), jnp.bfloat16))
    return pl.pallas_call(
        _softmax_body,
        grid=(n_rows // _BLOCK_ROWS,),
        in_specs=[
            pl.BlockSpec((_BLOCK_ROWS, n_feat), lambda i: (i, 0)),
            pl.BlockSpec((n_feat, n_feat), lambda i: (0, 0)),
        ],
        out_specs=pl.BlockSpec((_BLOCK_ROWS, n_feat), lambda i: (i, 0)),
        out_shape=jax.ShapeDtypeStruct((n_rows, n_feat), jnp.float32),
        compiler_params=pltpu.CompilerParams(
            dimension_semantics=("arbitrary",),
            vmem_limit_bytes=56 * 1024 * 1024,
            flags={"XLA_TPU_STORE_TO_LOAD_FORWARDING_WINDOW": 12288},
        ),
    )(x, u)


# speedup vs baseline: 1.0017x; 1.0017x over previous
import jax
import jax.numpy as jnp
from jax.experimental import pallas as pl
from jax.experimental.pallas import tpu as pltpu

_BLOCK_ROWS = 1024
_STRIP = 1024


_SC_MIN = 3.0517578125e-05


def _pow2_ceil_scale(a):
    b = jax.lax.bitcast_convert_type(a, jnp.int32)
    p2 = jax.lax.bitcast_convert_type(
        (b + 0x7FFFFF) & jnp.int32(-8388608), jnp.float32)
    return jnp.clip(p2 * (1.0 / 128.0), _SC_MIN, 1.0)


def _pow2_recip(s):
    b = jax.lax.bitcast_convert_type(s, jnp.int32)
    return jax.lax.bitcast_convert_type(jnp.int32(0x7F000000) - b, jnp.float32)


def _strip_compute(x, u):
    ax = jnp.abs(x)
    scale = _pow2_ceil_scale(ax)
    m = jnp.clip(jnp.round(x * _pow2_recip(scale)), -128.0, 127.0)
    qx = m * scale

    new_mx = jnp.clip(jnp.floor(qx * 184.0), -1024.0, 1023.0) * (1 / 128.0)
    n = jnp.floor(new_mx)
    r = new_mx - n
    mexp = jnp.round(jnp.exp2(r) * 64.0)

    p = jnp.exp2(n * 11.0).astype(jnp.bfloat16)
    s = jnp.dot(p, u, preferred_element_type=jnp.float32)
    ebf = (jax.lax.bitcast_convert_type(s, jnp.int32) >> 23).astype(jnp.float32)
    M4 = jnp.floor((ebf - 170.5) * (1.0 / 11.0))
    m_fin = jnp.max(n, axis=1, keepdims=True)
    m_fin4 = m_fin - 4.0

    d1 = jnp.exp2(n - M4)
    d2 = jnp.exp2(M4 - m_fin4)
    t = jnp.floor(mexp * d1)
    v = t * d2
    th = jnp.floor(v)
    tl = v - th
    half = x.shape[1] // 2
    s_hi = jnp.sum(th, axis=1, keepdims=True)
    lo1 = jnp.sum(tl[:, :half], axis=1, keepdims=True)
    lo2 = jnp.sum(tl[:, half:], axis=1, keepdims=True)
    s_lo = ((lo1 * 32768.0).astype(jnp.int32)
            + (lo2 * 32768.0).astype(jnp.int32))
    mexp_sum = s_hi + (s_lo >> 15).astype(jnp.float32)

    rdiv = 4096.0 / mexp_sum
    mout = jnp.floor(mexp * rdiv)
    qout = mout * d1 * d2 * (1.0 / 4096.0)
    sc1 = _pow2_ceil_scale(qout)
    m1 = jnp.clip(jnp.round(qout * _pow2_recip(sc1)), -128.0, 127.0)
    q1 = m1 * sc1
    return jnp.where((m1 == 64.0) & (sc1 > _SC_MIN), q1 * 0.9921875, q1)


def _softmax_body(x_ref, u_ref, o_ref):
    u = u_ref[...]
    for i in range(_BLOCK_ROWS // _STRIP):
        sl = slice(i * _STRIP, (i + 1) * _STRIP)
        o_ref[sl, :] = _strip_compute(x_ref[sl, :], u)


def kernel(x):
    n_rows, n_feat = x.shape
    u = jnp.triu(jnp.ones((n_feat, n_feat), jnp.bfloat16))
    return pl.pallas_call(
        _softmax_body,
        grid=(n_rows // _BLOCK_ROWS,),
        in_specs=[
            pl.BlockSpec((_BLOCK_ROWS, n_feat), lambda i: (i, 0)),
            pl.BlockSpec((n_feat, n_feat), lambda i: (0, 0)),
        ],
        out_specs=pl.BlockSpec((_BLOCK_ROWS, n_feat), lambda i: (i, 0)),
        out_shape=jax.ShapeDtypeStruct((n_rows, n_feat), jnp.float32),
        compiler_params=pltpu.CompilerParams(
            dimension_semantics=("arbitrary",),
            vmem_limit_bytes=56 * 1024 * 1024,
        ),
    )(x, u)
